# Initial kernel scaffold; baseline (speedup 1.0000x reference)
import jax
import jax.numpy as jnp
from jax.experimental import pallas as pl


def kernel(x, edge_index, batch, W1, b1, W2, b2, Wfc, bfc):
    raise NotImplementedError("write your pallas kernel here")


# speedup vs baseline: 5.6758x; 5.6758x over previous
import functools

import jax
import jax.numpy as jnp
from jax import lax
from jax.experimental import pallas as pl
from jax.experimental.pallas import tpu as pltpu
from jax.experimental.pallas import tpu_sc as plsc

NSUB = 16
NCORE = 2
CHUNK = 128
GRP = 32


def _make_sc_scatter(n_tab, acc_rows, rpt, chunks, hw):
    mesh = plsc.VectorSubcoreMesh(core_axis_name="c", subcore_axis_name="s")

    @functools.partial(
        pl.kernel,
        mesh=mesh,
        out_type=jax.ShapeDtypeStruct((NCORE, acc_rows, hw), jnp.float32),
        scratch_types=[
            pltpu.VMEM((GRP, CHUNK), jnp.int32),
            pltpu.VMEM((GRP, CHUNK), jnp.int32),
            pltpu.VMEM((CHUNK, hw), jnp.float32),
            pltpu.VMEM_SHARED((acc_rows, hw), jnp.float32),
            pltpu.SemaphoreType.DMA,
        ],
    )
    def scat(table, srcix, dstix, zeros, out, srcv, dstv, rows, acc, sem):
        c = lax.axis_index("c")
        s = lax.axis_index("s")
        pltpu.sync_copy(zeros, acc.at[pl.ds(s * rpt, rpt)])
        plsc.subcore_barrier()

        def group(g, carry):
            pltpu.sync_copy(srcix.at[c, s, pl.ds(g * GRP, GRP)], srcv)
            pltpu.sync_copy(dstix.at[s, pl.ds(g * GRP, GRP)], dstv)

            def body(j, carry2):
                pltpu.async_copy(table.at[srcv.at[j]], rows, sem).wait()
                pltpu.sync_copy(rows, acc.at[dstv.at[j]], add=True)
                return carry2

            lax.fori_loop(0, GRP, body, 0)
            return carry

        lax.fori_loop(0, chunks // GRP, group, 0)
        plsc.subcore_barrier()
        pltpu.sync_copy(acc.at[pl.ds(s * rpt, rpt)],
                        out.at[c, pl.ds(s * rpt, rpt)])

    return scat


def _tc1(x, w1, deg, bn):
    n, d = x.shape
    hw = w1.shape[1] // 2

    def body(x_ref, w_ref, deg_ref, y_ref):
        dinv = 1.0 / jnp.sqrt(deg_ref[...] + 1.0)
        xw = jnp.dot(x_ref[...], w_ref[...], preferred_element_type=jnp.float32)
        y_ref[...] = (dinv * xw)[None]

    return pl.pallas_call(
        body,
        grid=(2, n // bn),
        in_specs=[
            pl.BlockSpec((bn, d), lambda h, r: (r, 0)),
            pl.BlockSpec((d, hw), lambda h, r: (0, h)),
            pl.BlockSpec((bn, 1), lambda h, r: (r, 0)),
        ],
        out_specs=pl.BlockSpec((1, bn, hw), lambda h, r: (h, r, 0)),
        out_shape=jax.ShapeDtypeStruct((2, n, hw), jnp.float32),
    )(x, w1, deg)


def _tc2(s1, y1, deg, w2, b1, bn):
    n = y1.shape[1]
    hw = y1.shape[2]

    def body(s_ref, y_ref, deg_ref, w_ref, b_ref, o_ref):
        dinv = 1.0 / jnp.sqrt(deg_ref[...] + 1.0)
        a0 = jnp.maximum(dinv * (s_ref[0] + y_ref[0]) + b_ref[0], 0.0)
        a1 = jnp.maximum(dinv * (s_ref[1] + y_ref[1]) + b_ref[1], 0.0)
        w = w_ref[...]
        xw = (jnp.dot(a0, w[:hw], preferred_element_type=jnp.float32)
              + jnp.dot(a1, w[hw:], preferred_element_type=jnp.float32))
        o_ref[...] = (dinv * xw)[None]

    return pl.pallas_call(
        body,
        grid=(2, n // bn),
        in_specs=[
            pl.BlockSpec((2, bn, hw), lambda h, r: (0, r, 0)),
            pl.BlockSpec((2, bn, hw), lambda h, r: (0, r, 0)),
            pl.BlockSpec((bn, 1), lambda h, r: (r, 0)),
            pl.BlockSpec((2 * hw, hw), lambda h, r: (0, h)),
            pl.BlockSpec((2, hw), lambda h, r: (0, 0)),
        ],
        out_specs=pl.BlockSpec((1, bn, hw), lambda h, r: (h, r, 0)),
        out_shape=jax.ShapeDtypeStruct((2, n, hw), jnp.float32),
    )(s1, y1, deg, w2, b1)


def _tc3(s2, y2, deg, b2, wfc, bfc, bn):
    n = y2.shape[1]
    hw = y2.shape[2]

    def body(s_ref, y_ref, deg_ref, b_ref, wfc_ref, bfc_ref, o_ref):
        dinv = 1.0 / jnp.sqrt(deg_ref[...] + 1.0)
        h0 = jnp.maximum(dinv * (s_ref[0] + y_ref[0]) + b_ref[0], 0.0)
        h1 = jnp.maximum(dinv * (s_ref[1] + y_ref[1]) + b_ref[1], 0.0)
        logit = jnp.sum(h0 * wfc_ref[0] + h1 * wfc_ref[1], axis=1,
                        keepdims=True) + bfc_ref[0]
        o_ref[...] = jax.nn.sigmoid(logit)

    return pl.pallas_call(
        body,
        grid=(n // bn,),
        in_specs=[
            pl.BlockSpec((2, bn, hw), lambda r: (0, r, 0)),
            pl.BlockSpec((2, bn, hw), lambda r: (0, r, 0)),
            pl.BlockSpec((bn, 1), lambda r: (r, 0)),
            pl.BlockSpec((2, hw), lambda r: (0, 0)),
            pl.BlockSpec((2, hw), lambda r: (0, 0)),
            pl.BlockSpec(memory_space=pltpu.SMEM),
        ],
        out_specs=pl.BlockSpec((bn, 1), lambda r: (r, 0)),
        out_shape=jax.ShapeDtypeStruct((n, 1), jnp.float32),
    )(s2, y2, deg, b2, wfc, bfc)


def kernel(x, edge_index, batch, W1, b1, W2, b2, Wfc, bfc):
    n, d = x.shape
    h = W1.shape[1]
    e = edge_index.shape[1]
    hw = h // 2
    bn = 1000

    chunks = GRP * (-(-e // (NSUB * CHUNK * GRP)))
    e_pad = NSUB * chunks * CHUNK
    rpt = 8 * (-(-(n + 1) // (NSUB * 8)))
    acc_rows = NSUB * rpt

    src = edge_index[0]
    dst = edge_index[1]
    pad = e_pad - e
    src_p = jnp.concatenate([src, jnp.zeros((pad,), jnp.int32)])
    dst_p = jnp.concatenate([dst, jnp.full((pad,), n, jnp.int32)])
    dstix = dst_p.reshape(NSUB, chunks, CHUNK)
    srcix = jnp.stack([src_p, src_p + n]).reshape(NCORE, NSUB, chunks, CHUNK)
    zeros_hw = jnp.zeros((rpt, hw), jnp.float32)
    ones_tab = jnp.ones((2 * n, hw), jnp.float32)

    scat_row = _make_sc_scatter(2 * n, acc_rows, rpt, chunks, hw)

    degw = scat_row(ones_tab, srcix, dstix, zeros_hw)
    deg = degw[0, :n, 0:1]

    y1 = _tc1(x, W1, deg, bn)
    s1 = scat_row(y1.reshape(2 * n, hw), srcix, dstix, zeros_hw)
    y2 = _tc2(s1[:, :n], y1, deg, W2, b1.reshape(2, hw), bn)
    s2 = scat_row(y2.reshape(2 * n, hw), srcix, dstix, zeros_hw)
    out = _tc3(s2[:, :n], y2, deg, b2.reshape(2, hw),
               Wfc[:, 0].reshape(2, hw), bfc, bn)
    return out


# speedup vs baseline: 8.0455x; 1.4175x over previous
import functools

import jax
import jax.numpy as jnp
from jax import lax
from jax.experimental import pallas as pl
from jax.experimental.pallas import tpu as pltpu
from jax.experimental.pallas import tpu_sc as plsc

NSUB = 16
NCORE = 2
CHUNK = 128
GRP = 16


def _make_sc_scatter(n_tab, acc_rows, rpt, chunks, hw):
    mesh = plsc.VectorSubcoreMesh(core_axis_name="c", subcore_axis_name="s")

    @functools.partial(
        pl.kernel,
        mesh=mesh,
        out_type=jax.ShapeDtypeStruct((NCORE, acc_rows, hw), jnp.float32),
        scratch_types=[
            pltpu.VMEM((GRP, CHUNK), jnp.int32),
            pltpu.VMEM((GRP, CHUNK), jnp.int32),
            pltpu.VMEM((CHUNK, hw), jnp.float32),
            pltpu.VMEM((CHUNK, hw), jnp.float32),
            pltpu.VMEM_SHARED((acc_rows, hw), jnp.float32),
            pltpu.SemaphoreType.DMA,
            pltpu.SemaphoreType.DMA,
            pltpu.SemaphoreType.DMA,
            pltpu.SemaphoreType.DMA,
        ],
    )
    def scat(table, srcix, dstix, zeros, out, srcv, dstv, rows_a, rows_b,
             acc, sga, sgb, ssa, ssb):
        c = lax.axis_index("c")
        s = lax.axis_index("s")
        pltpu.sync_copy(zeros, acc.at[pl.ds(s * rpt, rpt)])
        plsc.subcore_barrier()

        def group(g, carry):
            pltpu.sync_copy(srcix.at[c, s, pl.ds(g * GRP, GRP)], srcv)
            pltpu.sync_copy(dstix.at[s, pl.ds(g * GRP, GRP)], dstv)
            pltpu.async_copy(table.at[srcv.at[0]], rows_a, sga)
            pltpu.async_copy(table.at[srcv.at[1]], rows_b, sgb)

            def pair(p, carry2):
                j0 = 2 * p
                j1 = j0 + 1
                pltpu.make_async_copy(table.at[srcv.at[j0]], rows_a, sga).wait()
                pltpu.async_copy(rows_a, acc.at[dstv.at[j0]], ssa, add=True)
                pltpu.make_async_copy(table.at[srcv.at[j1]], rows_b, sgb).wait()
                pltpu.async_copy(rows_b, acc.at[dstv.at[j1]], ssb, add=True)
                pltpu.make_async_copy(rows_a, acc.at[dstv.at[j0]], ssa).wait()
                pltpu.async_copy(table.at[srcv.at[j0 + 2]], rows_a, sga)
                pltpu.make_async_copy(rows_b, acc.at[dstv.at[j1]], ssb).wait()
                pltpu.async_copy(table.at[srcv.at[j1 + 2]], rows_b, sgb)
                return carry2

            lax.fori_loop(0, GRP // 2 - 1, pair, 0)
            pltpu.make_async_copy(table.at[srcv.at[GRP - 2]], rows_a, sga).wait()
            pltpu.async_copy(rows_a, acc.at[dstv.at[GRP - 2]], ssa, add=True)
            pltpu.make_async_copy(table.at[srcv.at[GRP - 1]], rows_b, sgb).wait()
            pltpu.async_copy(rows_b, acc.at[dstv.at[GRP - 1]], ssb, add=True)
            pltpu.make_async_copy(rows_a, acc.at[dstv.at[GRP - 2]], ssa).wait()
            pltpu.make_async_copy(rows_b, acc.at[dstv.at[GRP - 1]], ssb).wait()
            return carry

        lax.fori_loop(0, chunks // GRP, group, 0)
        plsc.subcore_barrier()
        pltpu.sync_copy(acc.at[pl.ds(s * rpt, rpt)],
                        out.at[c, pl.ds(s * rpt, rpt)])

    return scat


def _make_sc_degree(acc_rows, rpt, chunks, hw):
    mesh = plsc.VectorSubcoreMesh(core_axis_name="c", subcore_axis_name="s")
    half = chunks // NCORE

    @functools.partial(
        pl.kernel,
        mesh=mesh,
        out_type=jax.ShapeDtypeStruct((NCORE, acc_rows, hw), jnp.float32),
        scratch_types=[
            pltpu.VMEM((GRP, CHUNK), jnp.int32),
            pltpu.VMEM((CHUNK, hw), jnp.float32),
            pltpu.VMEM_SHARED((acc_rows, hw), jnp.float32),
            pltpu.SemaphoreType.DMA,
        ],
    )
    def degk(dstix, ones, zeros, out, dstv, ones_v, acc, sem):
        c = lax.axis_index("c")
        s = lax.axis_index("s")
        pltpu.sync_copy(zeros, acc.at[pl.ds(s * rpt, rpt)])
        pltpu.sync_copy(ones, ones_v)
        plsc.subcore_barrier()

        def group(g, carry):
            pltpu.sync_copy(dstix.at[s, pl.ds(c * half + g * GRP, GRP)], dstv)

            def fire(j, c2):
                pltpu.async_copy(ones_v, acc.at[dstv.at[j]], sem, add=True)
                return c2

            lax.fori_loop(0, GRP, fire, 0)

            def drain(j, c2):
                pltpu.make_async_copy(ones_v, acc.at[dstv.at[j]], sem).wait()
                return c2

            lax.fori_loop(0, GRP, drain, 0)
            return carry

        lax.fori_loop(0, half // GRP, group, 0)
        plsc.subcore_barrier()
        pltpu.sync_copy(acc.at[pl.ds(s * rpt, rpt)],
                        out.at[c, pl.ds(s * rpt, rpt)])

    return degk


def _tc1(x, w1, deg, bn):
    n, d = x.shape
    hw = w1.shape[1] // 2

    def body(x_ref, w_ref, deg_ref, y_ref):
        dinv = 1.0 / jnp.sqrt(deg_ref[0] + deg_ref[1] + 1.0)
        xw = jnp.dot(x_ref[...], w_ref[...], preferred_element_type=jnp.float32)
        y_ref[...] = (dinv * xw)[None]

    return pl.pallas_call(
        body,
        grid=(2, n // bn),
        in_specs=[
            pl.BlockSpec((bn, d), lambda h, r: (r, 0)),
            pl.BlockSpec((d, hw), lambda h, r: (0, h)),
            pl.BlockSpec((2, bn, 1), lambda h, r: (0, r, 0)),
        ],
        out_specs=pl.BlockSpec((1, bn, hw), lambda h, r: (h, r, 0)),
        out_shape=jax.ShapeDtypeStruct((2, n, hw), jnp.float32),
    )(x, w1, deg)


def _tc2(s1, y1, deg, w2, b1, bn):
    n = y1.shape[1]
    hw = y1.shape[2]

    def body(s_ref, y_ref, deg_ref, w_ref, b_ref, o_ref):
        dinv = 1.0 / jnp.sqrt(deg_ref[0] + deg_ref[1] + 1.0)
        a0 = jnp.maximum(dinv * (s_ref[0] + y_ref[0]) + b_ref[0], 0.0)
        a1 = jnp.maximum(dinv * (s_ref[1] + y_ref[1]) + b_ref[1], 0.0)
        w = w_ref[...]
        xw = (jnp.dot(a0, w[:hw], preferred_element_type=jnp.float32)
              + jnp.dot(a1, w[hw:], preferred_element_type=jnp.float32))
        o_ref[...] = (dinv * xw)[None]

    return pl.pallas_call(
        body,
        grid=(2, n // bn),
        in_specs=[
            pl.BlockSpec((2, bn, hw), lambda h, r: (0, r, 0)),
            pl.BlockSpec((2, bn, hw), lambda h, r: (0, r, 0)),
            pl.BlockSpec((2, bn, 1), lambda h, r: (0, r, 0)),
            pl.BlockSpec((2 * hw, hw), lambda h, r: (0, h)),
            pl.BlockSpec((2, hw), lambda h, r: (0, 0)),
        ],
        out_specs=pl.BlockSpec((1, bn, hw), lambda h, r: (h, r, 0)),
        out_shape=jax.ShapeDtypeStruct((2, n, hw), jnp.float32),
    )(s1, y1, deg, w2, b1)


def _tc3(s2, y2, deg, b2, wfc, bfc, bn):
    n = y2.shape[1]
    hw = y2.shape[2]

    def body(s_ref, y_ref, deg_ref, b_ref, wfc_ref, bfc_ref, o_ref):
        dinv = 1.0 / jnp.sqrt(deg_ref[0] + deg_ref[1] + 1.0)
        h0 = jnp.maximum(dinv * (s_ref[0] + y_ref[0]) + b_ref[0], 0.0)
        h1 = jnp.maximum(dinv * (s_ref[1] + y_ref[1]) + b_ref[1], 0.0)
        logit = jnp.sum(h0 * wfc_ref[0] + h1 * wfc_ref[1], axis=1,
                        keepdims=True) + bfc_ref[0]
        o_ref[...] = jax.nn.sigmoid(logit)

    return pl.pallas_call(
        body,
        grid=(n // bn,),
        in_specs=[
            pl.BlockSpec((2, bn, hw), lambda r: (0, r, 0)),
            pl.BlockSpec((2, bn, hw), lambda r: (0, r, 0)),
            pl.BlockSpec((2, bn, 1), lambda r: (0, r, 0)),
            pl.BlockSpec((2, hw), lambda r: (0, 0)),
            pl.BlockSpec((2, hw), lambda r: (0, 0)),
            pl.BlockSpec(memory_space=pltpu.SMEM),
        ],
        out_specs=pl.BlockSpec((bn, 1), lambda r: (r, 0)),
        out_shape=jax.ShapeDtypeStruct((n, 1), jnp.float32),
    )(s2, y2, deg, b2, wfc, bfc)


def kernel(x, edge_index, batch, W1, b1, W2, b2, Wfc, bfc):
    n, d = x.shape
    h = W1.shape[1]
    e = edge_index.shape[1]
    hw = h // 2
    bn = 1000

    chunks = NCORE * GRP * (-(-e // (NSUB * CHUNK * GRP * NCORE)))
    e_pad = NSUB * chunks * CHUNK
    rpt = 8 * (-(-(n + 1) // (NSUB * 8)))
    acc_rows = NSUB * rpt

    src = edge_index[0]
    dst = edge_index[1]
    pad = e_pad - e
    src_p = jnp.concatenate([src, jnp.zeros((pad,), jnp.int32)])
    dst_p = jnp.concatenate([dst, jnp.full((pad,), n, jnp.int32)])
    dstix = dst_p.reshape(NSUB, chunks, CHUNK)
    srcix = jnp.stack([src_p, src_p + n]).reshape(NCORE, NSUB, chunks, CHUNK)
    zeros_hw = jnp.zeros((rpt, hw), jnp.float32)
    ones_hbm = jnp.ones((CHUNK, hw), jnp.float32)

    scat_row = _make_sc_scatter(2 * n, acc_rows, rpt, chunks, hw)
    deg_k = _make_sc_degree(acc_rows, rpt, chunks, hw)

    degw = deg_k(dstix, ones_hbm, zeros_hw)
    deg = degw[:, :n, 0:1]

    y1 = _tc1(x, W1, deg, bn)
    s1 = scat_row(y1.reshape(2 * n, hw), srcix, dstix, zeros_hw)
    y2 = _tc2(s1[:, :n], y1, deg, W2, b1.reshape(2, hw), bn)
    s2 = scat_row(y2.reshape(2 * n, hw), srcix, dstix, zeros_hw)
    out = _tc3(s2[:, :n], y2, deg, b2.reshape(2, hw),
               Wfc[:, 0].reshape(2, hw), bfc, bn)
    return out


# speedup vs baseline: 8.2109x; 1.0206x over previous
import functools

import jax
import jax.numpy as jnp
from jax import lax
from jax.experimental import pallas as pl
from jax.experimental.pallas import tpu as pltpu
from jax.experimental.pallas import tpu_sc as plsc

NSUB = 16
NCORE = 2
CHUNK = 128
GRP = 16


def _make_sc_scatter(n_tab, acc_rows, rpt, chunks, hw):
    mesh = plsc.VectorSubcoreMesh(core_axis_name="c", subcore_axis_name="s")

    @functools.partial(
        pl.kernel,
        mesh=mesh,
        out_type=jax.ShapeDtypeStruct((NCORE, acc_rows, hw), jnp.float32),
        scratch_types=[
            pltpu.VMEM((GRP, CHUNK), jnp.int32),
            pltpu.VMEM((GRP, CHUNK), jnp.int32),
            pltpu.VMEM((CHUNK, hw), jnp.float32),
            pltpu.VMEM((CHUNK, hw), jnp.float32),
            pltpu.VMEM_SHARED((acc_rows, hw), jnp.float32),
            pltpu.SemaphoreType.DMA,
            pltpu.SemaphoreType.DMA,
            pltpu.SemaphoreType.DMA,
            pltpu.SemaphoreType.DMA,
        ],
    )
    def scat(table, srcix, dstix, zeros, out, srcv, dstv, rows_a, rows_b,
             acc, sga, sgb, ssa, ssb):
        c = lax.axis_index("c")
        s = lax.axis_index("s")
        pltpu.sync_copy(zeros, acc.at[pl.ds(s * rpt, rpt)])
        plsc.subcore_barrier()

        def group(g, carry):
            pltpu.sync_copy(srcix.at[c, s, pl.ds(g * GRP, GRP)], srcv)
            pltpu.sync_copy(dstix.at[s, pl.ds(g * GRP, GRP)], dstv)
            pltpu.async_copy(table.at[srcv.at[0]], rows_a, sga)
            pltpu.async_copy(table.at[srcv.at[1]], rows_b, sgb)

            def pair(p, carry2):
                j0 = 2 * p
                j1 = j0 + 1
                pltpu.make_async_copy(table.at[srcv.at[j0]], rows_a, sga).wait()
                pltpu.async_copy(rows_a, acc.at[dstv.at[j0]], ssa, add=True)
                pltpu.make_async_copy(table.at[srcv.at[j1]], rows_b, sgb).wait()
                pltpu.async_copy(rows_b, acc.at[dstv.at[j1]], ssb, add=True)
                pltpu.make_async_copy(rows_a, acc.at[dstv.at[j0]], ssa).wait()
                pltpu.async_copy(table.at[srcv.at[j0 + 2]], rows_a, sga)
                pltpu.make_async_copy(rows_b, acc.at[dstv.at[j1]], ssb).wait()
                pltpu.async_copy(table.at[srcv.at[j1 + 2]], rows_b, sgb)
                return carry2

            lax.fori_loop(0, GRP // 2 - 1, pair, 0)
            pltpu.make_async_copy(table.at[srcv.at[GRP - 2]], rows_a, sga).wait()
            pltpu.async_copy(rows_a, acc.at[dstv.at[GRP - 2]], ssa, add=True)
            pltpu.make_async_copy(table.at[srcv.at[GRP - 1]], rows_b, sgb).wait()
            pltpu.async_copy(rows_b, acc.at[dstv.at[GRP - 1]], ssb, add=True)
            pltpu.make_async_copy(rows_a, acc.at[dstv.at[GRP - 2]], ssa).wait()
            pltpu.make_async_copy(rows_b, acc.at[dstv.at[GRP - 1]], ssb).wait()
            return carry

        lax.fori_loop(0, chunks // GRP, group, 0)
        plsc.subcore_barrier()
        pltpu.sync_copy(acc.at[pl.ds(s * rpt, rpt)],
                        out.at[c, pl.ds(s * rpt, rpt)])

    return scat


def _make_sc_scatter_esplit(n_tab, acc_rows, rpt, chunks, hw):
    mesh = plsc.VectorSubcoreMesh(core_axis_name="c", subcore_axis_name="s")
    half = chunks // NCORE

    @functools.partial(
        pl.kernel,
        mesh=mesh,
        out_type=jax.ShapeDtypeStruct((NCORE, acc_rows, hw), jnp.float32),
        scratch_types=[
            pltpu.VMEM((GRP, CHUNK), jnp.int32),
            pltpu.VMEM((GRP, CHUNK), jnp.int32),
            pltpu.VMEM((CHUNK, hw), jnp.float32),
            pltpu.VMEM((CHUNK, hw), jnp.float32),
            pltpu.VMEM_SHARED((acc_rows, hw), jnp.float32),
            pltpu.SemaphoreType.DMA,
            pltpu.SemaphoreType.DMA,
            pltpu.SemaphoreType.DMA,
            pltpu.SemaphoreType.DMA,
        ],
    )
    def scat(table, srcix, dstix, zeros, out, srcv, dstv, rows_a, rows_b,
             acc, sga, sgb, ssa, ssb):
        c = lax.axis_index("c")
        s = lax.axis_index("s")
        pltpu.sync_copy(zeros, acc.at[pl.ds(s * rpt, rpt)])
        plsc.subcore_barrier()

        def group(g, carry):
            base = c * half + g * GRP
            pltpu.sync_copy(srcix.at[s, pl.ds(base, GRP)], srcv)
            pltpu.sync_copy(dstix.at[s, pl.ds(base, GRP)], dstv)
            pltpu.async_copy(table.at[srcv.at[0]], rows_a, sga)
            pltpu.async_copy(table.at[srcv.at[1]], rows_b, sgb)

            def pair(p, carry2):
                j0 = 2 * p
                j1 = j0 + 1
                pltpu.make_async_copy(table.at[srcv.at[j0]], rows_a, sga).wait()
                pltpu.async_copy(rows_a, acc.at[dstv.at[j0]], ssa, add=True)
                pltpu.make_async_copy(table.at[srcv.at[j1]], rows_b, sgb).wait()
                pltpu.async_copy(rows_b, acc.at[dstv.at[j1]], ssb, add=True)
                pltpu.make_async_copy(rows_a, acc.at[dstv.at[j0]], ssa).wait()
                pltpu.async_copy(table.at[srcv.at[j0 + 2]], rows_a, sga)
                pltpu.make_async_copy(rows_b, acc.at[dstv.at[j1]], ssb).wait()
                pltpu.async_copy(table.at[srcv.at[j1 + 2]], rows_b, sgb)
                return carry2

            lax.fori_loop(0, GRP // 2 - 1, pair, 0)
            pltpu.make_async_copy(table.at[srcv.at[GRP - 2]], rows_a, sga).wait()
            pltpu.async_copy(rows_a, acc.at[dstv.at[GRP - 2]], ssa, add=True)
            pltpu.make_async_copy(table.at[srcv.at[GRP - 1]], rows_b, sgb).wait()
            pltpu.async_copy(rows_b, acc.at[dstv.at[GRP - 1]], ssb, add=True)
            pltpu.make_async_copy(rows_a, acc.at[dstv.at[GRP - 2]], ssa).wait()
            pltpu.make_async_copy(rows_b, acc.at[dstv.at[GRP - 1]], ssb).wait()
            return carry

        lax.fori_loop(0, half // GRP, group, 0)
        plsc.subcore_barrier()
        pltpu.sync_copy(acc.at[pl.ds(s * rpt, rpt)],
                        out.at[c, pl.ds(s * rpt, rpt)])

    return scat


def _make_sc_degree(acc_rows, rpt, chunks, hw):
    mesh = plsc.VectorSubcoreMesh(core_axis_name="c", subcore_axis_name="s")
    half = chunks // NCORE

    @functools.partial(
        pl.kernel,
        mesh=mesh,
        out_type=jax.ShapeDtypeStruct((NCORE, acc_rows, hw), jnp.float32),
        scratch_types=[
            pltpu.VMEM((GRP, CHUNK), jnp.int32),
            pltpu.VMEM((CHUNK, hw), jnp.float32),
            pltpu.VMEM_SHARED((acc_rows, hw), jnp.float32),
            pltpu.SemaphoreType.DMA,
        ],
    )
    def degk(dstix, ones, zeros, out, dstv, ones_v, acc, sem):
        c = lax.axis_index("c")
        s = lax.axis_index("s")
        pltpu.sync_copy(zeros, acc.at[pl.ds(s * rpt, rpt)])
        pltpu.sync_copy(ones, ones_v)
        plsc.subcore_barrier()

        def group(g, carry):
            pltpu.sync_copy(dstix.at[s, pl.ds(c * half + g * GRP, GRP)], dstv)

            def fire(j, c2):
                pltpu.async_copy(ones_v, acc.at[dstv.at[j]], sem, add=True)
                return c2

            lax.fori_loop(0, GRP, fire, 0)

            def drain(j, c2):
                pltpu.make_async_copy(ones_v, acc.at[dstv.at[j]], sem).wait()
                return c2

            lax.fori_loop(0, GRP, drain, 0)
            return carry

        lax.fori_loop(0, half // GRP, group, 0)
        plsc.subcore_barrier()
        pltpu.sync_copy(acc.at[pl.ds(s * rpt, rpt)],
                        out.at[c, pl.ds(s * rpt, rpt)])

    return degk


def _tc0(x, deg, bn):
    n, d = x.shape

    def body(x_ref, deg_ref, z_ref):
        dinv = 1.0 / jnp.sqrt(deg_ref[0] + deg_ref[1] + 1.0)
        z_ref[...] = dinv * x_ref[...]

    return pl.pallas_call(
        body,
        grid=(n // bn,),
        in_specs=[
            pl.BlockSpec((bn, d), lambda r: (r, 0)),
            pl.BlockSpec((2, bn, 1), lambda r: (0, r, 0)),
        ],
        out_specs=pl.BlockSpec((bn, d), lambda r: (r, 0)),
        out_shape=jax.ShapeDtypeStruct((n, d), jnp.float32),
    )(x, deg)


def _tc1(sp, z, deg, w1, b1, bn):
    n, d = z.shape
    hw = w1.shape[1] // 2

    def body(s_ref, z_ref, deg_ref, w_ref, b_ref, o_ref):
        dinv = 1.0 / jnp.sqrt(deg_ref[0] + deg_ref[1] + 1.0)
        u = dinv * (s_ref[0] + s_ref[1] + z_ref[...])
        b = jnp.where(pl.program_id(0) == 0, b_ref[0], b_ref[1])
        h1 = jnp.maximum(
            jnp.dot(u, w_ref[...], preferred_element_type=jnp.float32)
            + b, 0.0)
        o_ref[...] = (dinv * h1)[None]

    return pl.pallas_call(
        body,
        grid=(2, n // bn),
        in_specs=[
            pl.BlockSpec((2, bn, d), lambda h, r: (0, r, 0)),
            pl.BlockSpec((bn, d), lambda h, r: (r, 0)),
            pl.BlockSpec((2, bn, 1), lambda h, r: (0, r, 0)),
            pl.BlockSpec((d, hw), lambda h, r: (0, h)),
            pl.BlockSpec((2, hw), lambda h, r: (0, 0)),
        ],
        out_specs=pl.BlockSpec((1, bn, hw), lambda h, r: (h, r, 0)),
        out_shape=jax.ShapeDtypeStruct((2, n, hw), jnp.float32),
    )(sp, z, deg, w1, b1)


def _tc2(s2, z2, deg, w2, b2, wfc, bfc, bn):
    n = z2.shape[1]
    hw = z2.shape[2]

    def body(s_ref, z_ref, deg_ref, w_ref, b_ref, wfc_ref, bfc_ref, o_ref):
        dinv = 1.0 / jnp.sqrt(deg_ref[0] + deg_ref[1] + 1.0)
        u0 = dinv * (s_ref[0] + z_ref[0])
        u1 = dinv * (s_ref[1] + z_ref[1])
        w = w_ref[...]
        h2 = jnp.maximum(
            jnp.dot(u0, w[:hw], preferred_element_type=jnp.float32)
            + jnp.dot(u1, w[hw:], preferred_element_type=jnp.float32)
            + b_ref[0], 0.0)
        logit = jnp.sum(h2 * wfc_ref[0], axis=1, keepdims=True) + bfc_ref[0]
        o_ref[...] = jax.nn.sigmoid(logit)

    return pl.pallas_call(
        body,
        grid=(n // bn,),
        in_specs=[
            pl.BlockSpec((2, bn, hw), lambda r: (0, r, 0)),
            pl.BlockSpec((2, bn, hw), lambda r: (0, r, 0)),
            pl.BlockSpec((2, bn, 1), lambda r: (0, r, 0)),
            pl.BlockSpec((2 * hw, 2 * hw), lambda r: (0, 0)),
            pl.BlockSpec((1, 2 * hw), lambda r: (0, 0)),
            pl.BlockSpec((1, 2 * hw), lambda r: (0, 0)),
            pl.BlockSpec(memory_space=pltpu.SMEM),
        ],
        out_specs=pl.BlockSpec((bn, 1), lambda r: (r, 0)),
        out_shape=jax.ShapeDtypeStruct((n, 1), jnp.float32),
    )(s2, z2, deg, w2, b2, wfc, bfc)


def kernel(x, edge_index, batch, W1, b1, W2, b2, Wfc, bfc):
    n, d = x.shape
    h = W1.shape[1]
    e = edge_index.shape[1]
    hw = h // 2
    bn = 1000

    chunks = NCORE * GRP * (-(-e // (NSUB * CHUNK * GRP * NCORE)))
    e_pad = NSUB * chunks * CHUNK
    rpt = 8 * (-(-(n + 1) // (NSUB * 8)))
    acc_rows = NSUB * rpt

    src = edge_index[0]
    dst = edge_index[1]
    pad = e_pad - e
    src_p = jnp.concatenate([src, jnp.zeros((pad,), jnp.int32)])
    dst_p = jnp.concatenate([dst, jnp.full((pad,), n, jnp.int32)])
    dstix = dst_p.reshape(NSUB, chunks, CHUNK)
    srcix1 = src_p.reshape(NSUB, chunks, CHUNK)
    srcix2 = jnp.stack([src_p, src_p + n]).reshape(NCORE, NSUB, chunks, CHUNK)
    zeros_hw = jnp.zeros((rpt, hw), jnp.float32)
    ones_hbm = jnp.ones((CHUNK, hw), jnp.float32)

    scat_x = _make_sc_scatter_esplit(n, acc_rows, rpt, chunks, d)
    scat_row = _make_sc_scatter(2 * n, acc_rows, rpt, chunks, hw)
    deg_k = _make_sc_degree(acc_rows, rpt, chunks, hw)

    degw = deg_k(dstix, ones_hbm, zeros_hw)
    deg = degw[:, :n, 0:1]

    z = _tc0(x, deg, bn)
    sp = scat_x(z, srcix1, dstix, zeros_hw)
    z2 = _tc1(sp[:, :n], z, deg, W1, b1.reshape(2, hw), bn)
    s2 = scat_row(z2.reshape(2 * n, hw), srcix2, dstix, zeros_hw)
    out = _tc2(s2[:, :n], z2, deg, W2, b2.reshape(1, h),
               Wfc[:, 0].reshape(1, h), bfc, bn)
    return out


# speedup vs baseline: 9.1895x; 1.1192x over previous
import functools

import jax
import jax.numpy as jnp
from jax import lax
from jax.experimental import pallas as pl
from jax.experimental.pallas import tpu as pltpu
from jax.experimental.pallas import tpu_sc as plsc

NSUB = 16
NCORE = 2
CHUNK = 128
GRP = 16


def _make_sc_scatter(n_tab, acc_rows, rpt, chunks, hw):
    mesh = plsc.VectorSubcoreMesh(core_axis_name="c", subcore_axis_name="s")

    @functools.partial(
        pl.kernel,
        mesh=mesh,
        out_type=jax.ShapeDtypeStruct((NCORE, acc_rows, hw), jnp.float32),
        scratch_types=[
            pltpu.VMEM((GRP, CHUNK), jnp.int32),
            pltpu.VMEM((GRP, CHUNK), jnp.int32),
            pltpu.VMEM((CHUNK, hw), jnp.float32),
            pltpu.VMEM((CHUNK, hw), jnp.float32),
            pltpu.VMEM_SHARED((acc_rows, hw), jnp.float32),
            pltpu.SemaphoreType.DMA,
            pltpu.SemaphoreType.DMA,
            pltpu.SemaphoreType.DMA,
            pltpu.SemaphoreType.DMA,
        ],
    )
    def scat(table, srcix, dstix, zeros, out, srcv, dstv, rows_a, rows_b,
             acc, sga, sgb, ssa, ssb):
        c = lax.axis_index("c")
        s = lax.axis_index("s")
        pltpu.sync_copy(zeros, acc.at[pl.ds(s * rpt, rpt)])
        plsc.subcore_barrier()

        def group(g, carry):
            pltpu.sync_copy(srcix.at[c, s, pl.ds(g * GRP, GRP)], srcv)
            pltpu.sync_copy(dstix.at[s, pl.ds(g * GRP, GRP)], dstv)
            pltpu.async_copy(table.at[srcv.at[0]], rows_a, sga)
            pltpu.async_copy(table.at[srcv.at[1]], rows_b, sgb)

            def pair(p, carry2):
                j0 = 2 * p
                j1 = j0 + 1
                pltpu.make_async_copy(table.at[srcv.at[j0]], rows_a, sga).wait()
                pltpu.async_copy(rows_a, acc.at[dstv.at[j0]], ssa, add=True)
                pltpu.make_async_copy(table.at[srcv.at[j1]], rows_b, sgb).wait()
                pltpu.async_copy(rows_b, acc.at[dstv.at[j1]], ssb, add=True)
                pltpu.make_async_copy(rows_a, acc.at[dstv.at[j0]], ssa).wait()
                pltpu.async_copy(table.at[srcv.at[j0 + 2]], rows_a, sga)
                pltpu.make_async_copy(rows_b, acc.at[dstv.at[j1]], ssb).wait()
                pltpu.async_copy(table.at[srcv.at[j1 + 2]], rows_b, sgb)
                return carry2

            lax.fori_loop(0, GRP // 2 - 1, pair, 0)
            pltpu.make_async_copy(table.at[srcv.at[GRP - 2]], rows_a, sga).wait()
            pltpu.async_copy(rows_a, acc.at[dstv.at[GRP - 2]], ssa, add=True)
            pltpu.make_async_copy(table.at[srcv.at[GRP - 1]], rows_b, sgb).wait()
            pltpu.async_copy(rows_b, acc.at[dstv.at[GRP - 1]], ssb, add=True)
            pltpu.make_async_copy(rows_a, acc.at[dstv.at[GRP - 2]], ssa).wait()
            pltpu.make_async_copy(rows_b, acc.at[dstv.at[GRP - 1]], ssb).wait()
            return carry

        lax.fori_loop(0, chunks // GRP, group, 0)
        plsc.subcore_barrier()
        pltpu.sync_copy(acc.at[pl.ds(s * rpt, rpt)],
                        out.at[c, pl.ds(s * rpt, rpt)])

    return scat


def _make_sc_scatter_esplit(n_tab, acc_rows, rpt, chunks, hw):
    mesh = plsc.VectorSubcoreMesh(core_axis_name="c", subcore_axis_name="s")
    half = chunks // NCORE

    @functools.partial(
        pl.kernel,
        mesh=mesh,
        out_type=jax.ShapeDtypeStruct((NCORE, acc_rows, hw), jnp.float32),
        scratch_types=[
            pltpu.VMEM((GRP, CHUNK), jnp.int32),
            pltpu.VMEM((GRP, CHUNK), jnp.int32),
            pltpu.VMEM((CHUNK, hw), jnp.float32),
            pltpu.VMEM((CHUNK, hw), jnp.float32),
            pltpu.VMEM_SHARED((acc_rows, hw), jnp.float32),
            pltpu.SemaphoreType.DMA,
            pltpu.SemaphoreType.DMA,
            pltpu.SemaphoreType.DMA,
            pltpu.SemaphoreType.DMA,
        ],
    )
    def scat(table, srcix, dstix, zeros, out, srcv, dstv, rows_a, rows_b,
             acc, sga, sgb, ssa, ssb):
        c = lax.axis_index("c")
        s = lax.axis_index("s")
        pltpu.sync_copy(zeros, acc.at[pl.ds(s * rpt, rpt)])
        plsc.subcore_barrier()

        def group(g, carry):
            base = c * half + g * GRP
            pltpu.sync_copy(srcix.at[c, s, pl.ds(base, GRP)], srcv)
            pltpu.sync_copy(dstix.at[s, pl.ds(base, GRP)], dstv)
            pltpu.async_copy(table.at[srcv.at[0]], rows_a, sga)
            pltpu.async_copy(table.at[srcv.at[1]], rows_b, sgb)

            def pair(p, carry2):
                j0 = 2 * p
                j1 = j0 + 1
                pltpu.make_async_copy(table.at[srcv.at[j0]], rows_a, sga).wait()
                pltpu.async_copy(rows_a, acc.at[dstv.at[j0]], ssa, add=True)
                pltpu.make_async_copy(table.at[srcv.at[j1]], rows_b, sgb).wait()
                pltpu.async_copy(rows_b, acc.at[dstv.at[j1]], ssb, add=True)
                pltpu.make_async_copy(rows_a, acc.at[dstv.at[j0]], ssa).wait()
                pltpu.async_copy(table.at[srcv.at[j0 + 2]], rows_a, sga)
                pltpu.make_async_copy(rows_b, acc.at[dstv.at[j1]], ssb).wait()
                pltpu.async_copy(table.at[srcv.at[j1 + 2]], rows_b, sgb)
                return carry2

            lax.fori_loop(0, GRP // 2 - 1, pair, 0)
            pltpu.make_async_copy(table.at[srcv.at[GRP - 2]], rows_a, sga).wait()
            pltpu.async_copy(rows_a, acc.at[dstv.at[GRP - 2]], ssa, add=True)
            pltpu.make_async_copy(table.at[srcv.at[GRP - 1]], rows_b, sgb).wait()
            pltpu.async_copy(rows_b, acc.at[dstv.at[GRP - 1]], ssb, add=True)
            pltpu.make_async_copy(rows_a, acc.at[dstv.at[GRP - 2]], ssa).wait()
            pltpu.make_async_copy(rows_b, acc.at[dstv.at[GRP - 1]], ssb).wait()
            return carry

        lax.fori_loop(0, half // GRP, group, 0)
        plsc.subcore_barrier()
        pltpu.sync_copy(acc.at[pl.ds(s * rpt, rpt)],
                        out.at[c, pl.ds(s * rpt, rpt)])

    return scat


def _make_sc_degree(acc_rows, rpt, chunks, hw):
    mesh = plsc.VectorSubcoreMesh(core_axis_name="c", subcore_axis_name="s")
    half = chunks // NCORE

    @functools.partial(
        pl.kernel,
        mesh=mesh,
        out_type=jax.ShapeDtypeStruct((NCORE, acc_rows, hw), jnp.float32),
        scratch_types=[
            pltpu.VMEM((GRP, CHUNK), jnp.int32),
            pltpu.VMEM((CHUNK, hw), jnp.float32),
            pltpu.VMEM_SHARED((acc_rows, hw), jnp.float32),
            pltpu.SemaphoreType.DMA,
        ],
    )
    def degk(dstix, ones, zeros, out, dstv, ones_v, acc, sem):
        c = lax.axis_index("c")
        s = lax.axis_index("s")
        pltpu.sync_copy(zeros, acc.at[pl.ds(s * rpt, rpt)])
        pltpu.sync_copy(ones, ones_v)
        plsc.subcore_barrier()

        def group(g, carry):
            pltpu.sync_copy(dstix.at[s, pl.ds(c * half + g * GRP, GRP)], dstv)

            def fire(j, c2):
                pltpu.async_copy(ones_v, acc.at[dstv.at[j]], sem, add=True)
                return c2

            lax.fori_loop(0, GRP, fire, 0)

            def drain(j, c2):
                pltpu.make_async_copy(ones_v, acc.at[dstv.at[j]], sem).wait()
                return c2

            lax.fori_loop(0, GRP, drain, 0)
            return carry

        lax.fori_loop(0, half // GRP, group, 0)
        plsc.subcore_barrier()
        pltpu.sync_copy(acc.at[pl.ds(s * rpt, rpt)],
                        out.at[c, pl.ds(s * rpt, rpt)])

    return degk


def _tc0(x, deg, bn):
    n, d = x.shape
    nb = n // bn

    def body(x_ref, deg_ref, z_ref):
        dinv = 1.0 / jnp.sqrt(deg_ref[0] + deg_ref[1] + 1.0)
        z_ref[...] = dinv * x_ref[...]

    return pl.pallas_call(
        body,
        grid=(2, nb),
        in_specs=[
            pl.BlockSpec((bn, d), lambda h, r: (r, 0)),
            pl.BlockSpec((2, bn, 1), lambda h, r: (0, r, 0)),
        ],
        out_specs=pl.BlockSpec((bn, d), lambda h, r: (h * nb + r, 0)),
        out_shape=jax.ShapeDtypeStruct((2 * n, d), jnp.float32),
    )(x, deg)


def _tc1(sp, z, deg, w1, b1, bn):
    n, d = z.shape
    hw = w1.shape[1] // 2

    def body(s_ref, z_ref, deg_ref, w_ref, b_ref, o_ref):
        dinv = 1.0 / jnp.sqrt(deg_ref[0] + deg_ref[1] + 1.0)
        u = dinv * (s_ref[0] + s_ref[1] + z_ref[...])
        b = jnp.where(pl.program_id(0) == 0, b_ref[0], b_ref[1])
        h1 = jnp.maximum(
            jnp.dot(u, w_ref[...], preferred_element_type=jnp.float32)
            + b, 0.0)
        o_ref[...] = (dinv * h1)[None]

    return pl.pallas_call(
        body,
        grid=(2, n // bn),
        in_specs=[
            pl.BlockSpec((2, bn, d), lambda h, r: (0, r, 0)),
            pl.BlockSpec((bn, d), lambda h, r: (r, 0)),
            pl.BlockSpec((2, bn, 1), lambda h, r: (0, r, 0)),
            pl.BlockSpec((d, hw), lambda h, r: (0, h)),
            pl.BlockSpec((2, hw), lambda h, r: (0, 0)),
        ],
        out_specs=pl.BlockSpec((1, bn, hw), lambda h, r: (h, r, 0)),
        out_shape=jax.ShapeDtypeStruct((2, n, hw), jnp.float32),
    )(sp, z, deg, w1, b1)


def _tc2(s2, z2, deg, w2, b2, wfc, bfc, bn):
    n = z2.shape[1]
    hw = z2.shape[2]

    def body(s_ref, z_ref, deg_ref, w_ref, b_ref, wfc_ref, bfc_ref, o_ref):
        dinv = 1.0 / jnp.sqrt(deg_ref[0] + deg_ref[1] + 1.0)
        u0 = dinv * (s_ref[0] + z_ref[0])
        u1 = dinv * (s_ref[1] + z_ref[1])
        w = w_ref[...]
        h2 = jnp.maximum(
            jnp.dot(u0, w[:hw], preferred_element_type=jnp.float32)
            + jnp.dot(u1, w[hw:], preferred_element_type=jnp.float32)
            + b_ref[0], 0.0)
        logit = jnp.sum(h2 * wfc_ref[0], axis=1, keepdims=True) + bfc_ref[0]
        o_ref[...] = jax.nn.sigmoid(logit)

    return pl.pallas_call(
        body,
        grid=(n // bn,),
        in_specs=[
            pl.BlockSpec((2, bn, hw), lambda r: (0, r, 0)),
            pl.BlockSpec((2, bn, hw), lambda r: (0, r, 0)),
            pl.BlockSpec((2, bn, 1), lambda r: (0, r, 0)),
            pl.BlockSpec((2 * hw, 2 * hw), lambda r: (0, 0)),
            pl.BlockSpec((1, 2 * hw), lambda r: (0, 0)),
            pl.BlockSpec((1, 2 * hw), lambda r: (0, 0)),
            pl.BlockSpec(memory_space=pltpu.SMEM),
        ],
        out_specs=pl.BlockSpec((bn, 1), lambda r: (r, 0)),
        out_shape=jax.ShapeDtypeStruct((n, 1), jnp.float32),
    )(s2, z2, deg, w2, b2, wfc, bfc)


def kernel(x, edge_index, batch, W1, b1, W2, b2, Wfc, bfc):
    n, d = x.shape
    h = W1.shape[1]
    e = edge_index.shape[1]
    hw = h // 2
    bn = 1000

    chunks = NCORE * GRP * (-(-e // (NSUB * CHUNK * GRP * NCORE)))
    e_pad = NSUB * chunks * CHUNK
    rpt = 8 * (-(-(n + 1) // (NSUB * 8)))
    acc_rows = NSUB * rpt

    src = edge_index[0]
    dst = edge_index[1]
    pad = e_pad - e
    src_p = jnp.concatenate([src, jnp.zeros((pad,), jnp.int32)])
    dst_p = jnp.concatenate([dst, jnp.full((pad,), n, jnp.int32)])
    dstix = dst_p.reshape(NSUB, chunks, CHUNK)
    srcix2 = jnp.stack([src_p, src_p + n]).reshape(NCORE, NSUB, chunks, CHUNK)
    zeros_hw = jnp.zeros((rpt, hw), jnp.float32)
    ones_hbm = jnp.ones((CHUNK, hw), jnp.float32)

    scat_x = _make_sc_scatter_esplit(2 * n, acc_rows, rpt, chunks, d)
    scat_row = _make_sc_scatter(2 * n, acc_rows, rpt, chunks, hw)
    deg_k = _make_sc_degree(acc_rows, rpt, chunks, hw)

    degw = deg_k(dstix, ones_hbm, zeros_hw)
    deg = degw[:, :n, 0:1]

    zz = _tc0(x, deg, bn)
    sp = scat_x(zz, srcix2, dstix, zeros_hw)
    z2 = _tc1(sp[:, :n], zz[:n], deg, W1, b1.reshape(2, hw), bn)
    s2 = scat_row(z2.reshape(2 * n, hw), srcix2, dstix, zeros_hw)
    out = _tc2(s2[:, :n], z2, deg, W2, b2.reshape(1, h),
               Wfc[:, 0].reshape(1, h), bfc, bn)
    return out


# speedup vs baseline: 9.1950x; 1.0006x over previous
import functools

import jax
import jax.numpy as jnp
from jax import lax
from jax.experimental import pallas as pl
from jax.experimental.pallas import tpu as pltpu
from jax.experimental.pallas import tpu_sc as plsc

NSUB = 16
NCORE = 2
CHUNK = 128
GRP = 16


def _make_sc_scatter(n_tab, acc_rows, rpt, chunks, hw):
    mesh = plsc.VectorSubcoreMesh(core_axis_name="c", subcore_axis_name="s")

    @functools.partial(
        pl.kernel,
        mesh=mesh,
        out_type=jax.ShapeDtypeStruct((NCORE, acc_rows, hw), jnp.float32),
        scratch_types=[
            pltpu.VMEM((GRP, CHUNK), jnp.int32),
            pltpu.VMEM((GRP, CHUNK), jnp.int32),
            pltpu.VMEM((CHUNK, hw), jnp.float32),
            pltpu.VMEM((CHUNK, hw), jnp.float32),
            pltpu.VMEM_SHARED((acc_rows, hw), jnp.float32),
            pltpu.SemaphoreType.DMA,
            pltpu.SemaphoreType.DMA,
            pltpu.SemaphoreType.DMA,
            pltpu.SemaphoreType.DMA,
        ],
    )
    def scat(table, srcix, dstix, zeros, out, srcv, dstv, rows_a, rows_b,
             acc, sga, sgb, ssa, ssb):
        c = lax.axis_index("c")
        s = lax.axis_index("s")
        pltpu.sync_copy(zeros, acc.at[pl.ds(s * rpt, rpt)])
        plsc.subcore_barrier()

        def group(g, carry):
            pltpu.sync_copy(srcix.at[c, s, pl.ds(g * GRP, GRP)], srcv)
            pltpu.sync_copy(dstix.at[s, pl.ds(g * GRP, GRP)], dstv)
            pltpu.async_copy(table.at[srcv.at[0]], rows_a, sga)
            pltpu.async_copy(table.at[srcv.at[1]], rows_b, sgb)

            def pair(p, carry2):
                j0 = 2 * p
                j1 = j0 + 1
                pltpu.make_async_copy(table.at[srcv.at[j0]], rows_a, sga).wait()
                pltpu.async_copy(rows_a, acc.at[dstv.at[j0]], ssa, add=True)
                pltpu.make_async_copy(table.at[srcv.at[j1]], rows_b, sgb).wait()
                pltpu.async_copy(rows_b, acc.at[dstv.at[j1]], ssb, add=True)
                pltpu.make_async_copy(rows_a, acc.at[dstv.at[j0]], ssa).wait()
                pltpu.async_copy(table.at[srcv.at[j0 + 2]], rows_a, sga)
                pltpu.make_async_copy(rows_b, acc.at[dstv.at[j1]], ssb).wait()
                pltpu.async_copy(table.at[srcv.at[j1 + 2]], rows_b, sgb)
                return carry2

            lax.fori_loop(0, GRP // 2 - 1, pair, 0)
            pltpu.make_async_copy(table.at[srcv.at[GRP - 2]], rows_a, sga).wait()
            pltpu.async_copy(rows_a, acc.at[dstv.at[GRP - 2]], ssa, add=True)
            pltpu.make_async_copy(table.at[srcv.at[GRP - 1]], rows_b, sgb).wait()
            pltpu.async_copy(rows_b, acc.at[dstv.at[GRP - 1]], ssb, add=True)
            pltpu.make_async_copy(rows_a, acc.at[dstv.at[GRP - 2]], ssa).wait()
            pltpu.make_async_copy(rows_b, acc.at[dstv.at[GRP - 1]], ssb).wait()
            return carry

        lax.fori_loop(0, chunks // GRP, group, 0)
        plsc.subcore_barrier()
        pltpu.sync_copy(acc.at[pl.ds(s * rpt, rpt)],
                        out.at[c, pl.ds(s * rpt, rpt)])

    return scat


def _make_sc_scatter_esplit(n_tab, acc_rows, rpt, chunks, hw):
    mesh = plsc.VectorSubcoreMesh(core_axis_name="c", subcore_axis_name="s")
    half = chunks // NCORE

    @functools.partial(
        pl.kernel,
        mesh=mesh,
        out_type=jax.ShapeDtypeStruct((NCORE, acc_rows, hw), jnp.float32),
        scratch_types=[
            pltpu.VMEM((GRP, CHUNK), jnp.int32),
            pltpu.VMEM((GRP, CHUNK), jnp.int32),
            pltpu.VMEM((CHUNK, hw), jnp.float32),
            pltpu.VMEM((CHUNK, hw), jnp.float32),
            pltpu.VMEM_SHARED((acc_rows, hw), jnp.float32),
            pltpu.SemaphoreType.DMA,
            pltpu.SemaphoreType.DMA,
            pltpu.SemaphoreType.DMA,
            pltpu.SemaphoreType.DMA,
        ],
    )
    def scat(table, srcix, dstix, zeros, out, srcv, dstv, rows_a, rows_b,
             acc, sga, sgb, ssa, ssb):
        c = lax.axis_index("c")
        s = lax.axis_index("s")
        pltpu.sync_copy(zeros, acc.at[pl.ds(s * rpt, rpt)])
        plsc.subcore_barrier()

        def group(g, carry):
            base = c * half + g * GRP
            pltpu.sync_copy(srcix.at[c, s, pl.ds(base, GRP)], srcv)
            pltpu.sync_copy(dstix.at[s, pl.ds(base, GRP)], dstv)
            pltpu.async_copy(table.at[srcv.at[0]], rows_a, sga)
            pltpu.async_copy(table.at[srcv.at[1]], rows_b, sgb)

            def pair(p, carry2):
                j0 = 2 * p
                j1 = j0 + 1
                pltpu.make_async_copy(table.at[srcv.at[j0]], rows_a, sga).wait()
                pltpu.async_copy(rows_a, acc.at[dstv.at[j0]], ssa, add=True)
                pltpu.make_async_copy(table.at[srcv.at[j1]], rows_b, sgb).wait()
                pltpu.async_copy(rows_b, acc.at[dstv.at[j1]], ssb, add=True)
                pltpu.make_async_copy(rows_a, acc.at[dstv.at[j0]], ssa).wait()
                pltpu.async_copy(table.at[srcv.at[j0 + 2]], rows_a, sga)
                pltpu.make_async_copy(rows_b, acc.at[dstv.at[j1]], ssb).wait()
                pltpu.async_copy(table.at[srcv.at[j1 + 2]], rows_b, sgb)
                return carry2

            lax.fori_loop(0, GRP // 2 - 1, pair, 0)
            pltpu.make_async_copy(table.at[srcv.at[GRP - 2]], rows_a, sga).wait()
            pltpu.async_copy(rows_a, acc.at[dstv.at[GRP - 2]], ssa, add=True)
            pltpu.make_async_copy(table.at[srcv.at[GRP - 1]], rows_b, sgb).wait()
            pltpu.async_copy(rows_b, acc.at[dstv.at[GRP - 1]], ssb, add=True)
            pltpu.make_async_copy(rows_a, acc.at[dstv.at[GRP - 2]], ssa).wait()
            pltpu.make_async_copy(rows_b, acc.at[dstv.at[GRP - 1]], ssb).wait()
            return carry

        lax.fori_loop(0, half // GRP, group, 0)
        plsc.subcore_barrier()
        pltpu.sync_copy(acc.at[pl.ds(s * rpt, rpt)],
                        out.at[c, pl.ds(s * rpt, rpt)])

    return scat


def _make_sc_degree(acc_rows, rpt, chunks, hw):
    mesh = plsc.VectorSubcoreMesh(core_axis_name="c", subcore_axis_name="s")
    half = chunks // NCORE

    @functools.partial(
        pl.kernel,
        mesh=mesh,
        out_type=jax.ShapeDtypeStruct((NCORE, acc_rows, hw), jnp.float32),
        scratch_types=[
            pltpu.VMEM((GRP, CHUNK), jnp.int32),
            pltpu.VMEM((CHUNK, hw), jnp.float32),
            pltpu.VMEM_SHARED((acc_rows, hw), jnp.float32),
            pltpu.SemaphoreType.DMA,
        ],
    )
    def degk(dstix, ones, zeros, out, dstv, ones_v, acc, sem):
        c = lax.axis_index("c")
        s = lax.axis_index("s")
        pltpu.sync_copy(zeros, acc.at[pl.ds(s * rpt, rpt)])
        pltpu.sync_copy(ones, ones_v)
        plsc.subcore_barrier()

        def group(g, carry):
            pltpu.sync_copy(dstix.at[s, pl.ds(c * half + g * GRP, GRP)], dstv)

            def fire(j, c2):
                pltpu.async_copy(ones_v, acc.at[dstv.at[j]], sem, add=True)
                return c2

            lax.fori_loop(0, GRP, fire, 0)

            def drain(j, c2):
                pltpu.make_async_copy(ones_v, acc.at[dstv.at[j]], sem).wait()
                return c2

            lax.fori_loop(0, GRP, drain, 0)
            return carry

        lax.fori_loop(0, half // GRP, group, 0)
        plsc.subcore_barrier()
        pltpu.sync_copy(acc.at[pl.ds(s * rpt, rpt)],
                        out.at[c, pl.ds(s * rpt, rpt)])

    return degk


def _tc0(x, deg, bn):
    n, d = x.shape
    nb = n // bn

    def body(x_ref, deg_ref, z_ref):
        dinv = 1.0 / jnp.sqrt(deg_ref[0] + deg_ref[1] + 1.0)
        z_ref[...] = dinv * x_ref[...]

    return pl.pallas_call(
        body,
        grid=(2, nb),
        in_specs=[
            pl.BlockSpec((bn, d), lambda h, r: (r, 0)),
            pl.BlockSpec((2, bn, 1), lambda h, r: (0, r, 0)),
        ],
        out_specs=pl.BlockSpec((bn, d), lambda h, r: (h * nb + r, 0)),
        out_shape=jax.ShapeDtypeStruct((2 * n, d), jnp.float32),
    )(x, deg)


def _tc1(sp, z, deg, w1, b1, bn):
    n, d = z.shape
    hw = w1.shape[1] // 2

    def body(s_ref, z_ref, deg_ref, w_ref, b_ref, o_ref):
        dinv = 1.0 / jnp.sqrt(deg_ref[0] + deg_ref[1] + 1.0)
        u = dinv * (s_ref[0] + s_ref[1] + z_ref[...])
        b = jnp.where(pl.program_id(0) == 0, b_ref[0], b_ref[1])
        h1 = jnp.maximum(
            jnp.dot(u, w_ref[...], preferred_element_type=jnp.float32)
            + b, 0.0)
        o_ref[...] = (dinv * h1)[None]

    return pl.pallas_call(
        body,
        grid=(2, n // bn),
        in_specs=[
            pl.BlockSpec((2, bn, d), lambda h, r: (0, r, 0)),
            pl.BlockSpec((bn, d), lambda h, r: (r, 0)),
            pl.BlockSpec((2, bn, 1), lambda h, r: (0, r, 0)),
            pl.BlockSpec((d, hw), lambda h, r: (0, h)),
            pl.BlockSpec((2, hw), lambda h, r: (0, 0)),
        ],
        out_specs=pl.BlockSpec((1, bn, hw), lambda h, r: (h, r, 0)),
        out_shape=jax.ShapeDtypeStruct((2, n, hw), jnp.float32),
    )(sp, z, deg, w1, b1)


def _tc2(s2, z2, deg, w2, b2, wfc, bfc, bn):
    n = z2.shape[1]
    hw = z2.shape[2]

    def body(s_ref, z_ref, deg_ref, w_ref, b_ref, wfc_ref, bfc_ref, o_ref):
        dinv = 1.0 / jnp.sqrt(deg_ref[0] + deg_ref[1] + 1.0)
        u0 = dinv * (s_ref[0] + z_ref[0].astype(jnp.float32))
        u1 = dinv * (s_ref[1] + z_ref[1].astype(jnp.float32))
        w = w_ref[...]
        h2 = jnp.maximum(
            jnp.dot(u0, w[:hw], preferred_element_type=jnp.float32)
            + jnp.dot(u1, w[hw:], preferred_element_type=jnp.float32)
            + b_ref[0], 0.0)
        logit = jnp.sum(h2 * wfc_ref[0], axis=1, keepdims=True) + bfc_ref[0]
        o_ref[...] = jax.nn.sigmoid(logit)

    return pl.pallas_call(
        body,
        grid=(n // bn,),
        in_specs=[
            pl.BlockSpec((2, bn, hw), lambda r: (0, r, 0)),
            pl.BlockSpec((2, bn, hw), lambda r: (0, r, 0)),
            pl.BlockSpec((2, bn, 1), lambda r: (0, r, 0)),
            pl.BlockSpec((2 * hw, 2 * hw), lambda r: (0, 0)),
            pl.BlockSpec((1, 2 * hw), lambda r: (0, 0)),
            pl.BlockSpec((1, 2 * hw), lambda r: (0, 0)),
            pl.BlockSpec(memory_space=pltpu.SMEM),
        ],
        out_specs=pl.BlockSpec((bn, 1), lambda r: (r, 0)),
        out_shape=jax.ShapeDtypeStruct((n, 1), jnp.float32),
    )(s2, z2, deg, w2, b2, wfc, bfc)


def kernel(x, edge_index, batch, W1, b1, W2, b2, Wfc, bfc):
    n, d = x.shape
    h = W1.shape[1]
    e = edge_index.shape[1]
    hw = h // 2
    bn = 1000

    chunks = NCORE * GRP * (-(-e // (NSUB * CHUNK * GRP * NCORE)))
    e_pad = NSUB * chunks * CHUNK
    rpt = 8 * (-(-(n + 1) // (NSUB * 8)))
    acc_rows = NSUB * rpt

    src = edge_index[0]
    dst = edge_index[1]
    pad = e_pad - e
    src_p = jnp.concatenate([src, jnp.zeros((pad,), jnp.int32)])
    dst_p = jnp.concatenate([dst, jnp.full((pad,), n, jnp.int32)])
    dstix = dst_p.reshape(NSUB, chunks, CHUNK)
    srcix2 = jnp.stack([src_p, src_p + n]).reshape(NCORE, NSUB, chunks, CHUNK)
    zeros_hw = jnp.zeros((rpt, hw), jnp.float32)
    ones_hbm = jnp.ones((CHUNK, hw), jnp.float32)

    scat_x = _make_sc_scatter_esplit(2 * n, acc_rows, rpt, chunks, d)
    scat_row = _make_sc_scatter(2 * n, acc_rows, rpt, chunks, hw)
    deg_k = _make_sc_degree(acc_rows, rpt, chunks, hw)

    degw = deg_k(dstix, ones_hbm, zeros_hw)
    deg = degw[:, :n, 0:1]

    zz = _tc0(x, deg, bn)
    sp = scat_x(zz, srcix2, dstix, zeros_hw)
    z2 = _tc1(sp[:, :n], zz[:n], deg, W1, b1.reshape(2, hw), bn)
    s2 = scat_row(z2.reshape(2 * n, hw), srcix2, dstix, zeros_hw)

    out = _tc2(s2[:, :n], z2, deg, W2, b2.reshape(1, h),
               Wfc[:, 0].reshape(1, h), bfc, bn)
    return out
